# Initial kernel scaffold; baseline (speedup 1.0000x reference)
import jax
import jax.numpy as jnp
from jax.experimental import pallas as pl


def kernel(pos, edge_index, batch, W0, b0, W1, b1):
    raise NotImplementedError("write your pallas kernel here")


# speedup vs baseline: 83.0941x; 83.0941x over previous
import functools

import jax
import jax.numpy as jnp
from jax import lax
from jax.experimental import pallas as pl
from jax.experimental.pallas import tpu as pltpu
from jax.experimental.pallas import tpu_sc as plsc

N_NODES = 50000
N_EDGES = 1600000
NUM_GRAPHS = 16
IN_CH = 3
HIDDEN = 64
OUT_CH = 24

NPAD = 51200
NROW = NPAD // 128
NCORES = 2
NSUB = 16
NTILES = NCORES * NSUB
C = 1024
CHUNKS = 51
E_TILE = C * CHUNKS
E_TOTAL = E_TILE * NTILES
NP16 = NPAD // NSUB
W2SZ = NUM_GRAPHS * NPAD
W2T = W2SZ // NSUB
RW = 8

_MESH = dict(core_axis_name="c", subcore_axis_name="s",
             num_cores=NCORES, num_subcores=NSUB)
_SC_PARAMS = pltpu.CompilerParams(needs_layout_passes=False,
                                  use_tc_tiling_on_sc=False)


def _wid():
    return lax.axis_index("c") * NSUB + lax.axis_index("s")


def _k1_body(dst_hbm, zw_hbm, deg_out, dst_v, ones_v, deg_sh):
    cid = lax.axis_index("c")
    sid = lax.axis_index("s")
    w = _wid()
    for g in range(C // 16):
        ones_v[pl.ds(g * 16, 16)] = jnp.ones((16,), jnp.float32)
    pltpu.sync_copy(zw_hbm.at[pl.ds(0, NP16)],
                    deg_sh.at[pl.ds(sid * NP16, NP16)])
    plsc.subcore_barrier()

    def chunk(i, _):
        eb = pl.multiple_of(w * E_TILE + i * C, C)
        pltpu.sync_copy(dst_hbm.at[pl.ds(eb, C)], dst_v)
        pltpu.sync_copy(ones_v, deg_sh.at[dst_v], add=True)
        return ()

    lax.fori_loop(0, CHUNKS, chunk, ())
    plsc.subcore_barrier()
    pltpu.sync_copy(deg_sh.at[pl.ds(sid * NP16, NP16)],
                    deg_out.at[cid, pl.ds(sid * NP16, NP16)])


_k1 = functools.partial(
    pl.kernel,
    out_type=jax.ShapeDtypeStruct((NCORES, NPAD), jnp.float32),
    mesh=plsc.VectorSubcoreMesh(**_MESH),
    scratch_types=[
        pltpu.VMEM((C,), jnp.int32),
        pltpu.VMEM((C,), jnp.float32),
        pltpu.VMEM_SHARED((NPAD,), jnp.float32),
    ],
    compiler_params=_SC_PARAMS,
)(_k1_body)


def _k2_body(deg2_ref, batch_ref, posi_ref, dinv_ref, qi_ref, cnt_ref):
    deg = deg2_ref[0] + deg2_ref[1]
    dinv = jnp.where(deg > 0, lax.rsqrt(deg), 0.0)
    dinv_ref[...] = dinv
    qi_ref[...] = jnp.repeat(dinv, RW, axis=1) * posi_ref[...]
    b = batch_ref[...]
    nid = (lax.broadcasted_iota(jnp.int32, (NROW, 128), 0) * 128
           + lax.broadcasted_iota(jnp.int32, (NROW, 128), 1))
    real = nid < N_NODES
    for g in range(NUM_GRAPHS):
        cnt_ref[0, g] = jnp.sum(jnp.where((b == g) & real, 1.0, 0.0))


def _k2(deg2, batch2d, posi):
    return pl.pallas_call(
        _k2_body,
        out_shape=[
            jax.ShapeDtypeStruct((NROW, 128), jnp.float32),
            jax.ShapeDtypeStruct((NROW, 128 * RW), jnp.float32),
            jax.ShapeDtypeStruct((1, NUM_GRAPHS), jnp.float32),
        ],
        out_specs=[
            pl.BlockSpec(memory_space=pltpu.VMEM),
            pl.BlockSpec(memory_space=pltpu.VMEM),
            pl.BlockSpec(memory_space=pltpu.SMEM),
        ],
    )(deg2, batch2d, posi)


def _k3a_body(src_hbm, dst_hbm, q_hbm, zs_hbm, s_out,
              src_v, dst_v, qrow_v, q_sh, s_sh):
    cid = lax.axis_index("c")
    sid = lax.axis_index("s")
    w = _wid()

    pltpu.sync_copy(q_hbm.at[pl.ds(sid * NP16, NP16)],
                    q_sh.at[pl.ds(sid * NP16, NP16)])
    for r in range(4):
        pltpu.sync_copy(zs_hbm,
                        s_sh.at[pl.ds(sid * NP16 + r * (NP16 // 4), NP16 // 4)])
    plsc.subcore_barrier()

    def chunk(i, _):
        eb = pl.multiple_of(w * E_TILE + i * C, C)
        pltpu.sync_copy(src_hbm.at[pl.ds(eb, C)], src_v)
        pltpu.sync_copy(dst_hbm.at[pl.ds(eb, C)], dst_v)
        pltpu.sync_copy(q_sh.at[src_v], qrow_v)
        pltpu.sync_copy(qrow_v, s_sh.at[dst_v], add=True)
        return ()

    lax.fori_loop(0, CHUNKS, chunk, ())
    plsc.subcore_barrier()
    pltpu.sync_copy(s_sh.at[pl.ds(sid * NP16, NP16)],
                    s_out.at[cid, pl.ds(sid * NP16, NP16)])


_k3a = functools.partial(
    pl.kernel,
    out_type=jax.ShapeDtypeStruct((NCORES, NPAD, RW), jnp.float32),
    mesh=plsc.VectorSubcoreMesh(**_MESH),
    scratch_types=[
        pltpu.VMEM((C,), jnp.int32),
        pltpu.VMEM((C,), jnp.int32),
        pltpu.VMEM((C, RW), jnp.float32),
        pltpu.VMEM_SHARED((NPAD, RW), jnp.float32),
        pltpu.VMEM_SHARED((NPAD, RW), jnp.float32),
    ],
    compiler_params=_SC_PARAMS,
)(_k3a_body)


def _k3b_body(src_hbm, dst_hbm, dinv_hbm, batch_hbm, zw_hbm, w2_out,
              src_v, dst_v, val_v, bt_v, idx_v, dinv_sh, batch_sh, w2_sh):
    cid = lax.axis_index("c")
    sid = lax.axis_index("s")
    w = _wid()

    pltpu.sync_copy(dinv_hbm.at[pl.ds(sid * NP16, NP16)],
                    dinv_sh.at[pl.ds(sid * NP16, NP16)])
    pltpu.sync_copy(batch_hbm.at[pl.ds(sid * NP16, NP16)],
                    batch_sh.at[pl.ds(sid * NP16, NP16)])
    pltpu.sync_copy(zw_hbm, w2_sh.at[pl.ds(sid * W2T, W2T)])
    plsc.subcore_barrier()

    def chunk(i, _):
        eb = pl.multiple_of(w * E_TILE + i * C, C)
        pltpu.sync_copy(src_hbm.at[pl.ds(eb, C)], src_v)
        pltpu.sync_copy(dst_hbm.at[pl.ds(eb, C)], dst_v)
        pltpu.sync_copy(dinv_sh.at[dst_v], val_v)
        pltpu.sync_copy(batch_sh.at[dst_v], bt_v)
        for g in range(C // 16):
            idx_v[pl.ds(g * 16, 16)] = (
                bt_v[pl.ds(g * 16, 16)] * NPAD + src_v[pl.ds(g * 16, 16)])
        pltpu.sync_copy(val_v, w2_sh.at[idx_v], add=True)
        return ()

    lax.fori_loop(0, CHUNKS, chunk, ())
    plsc.subcore_barrier()
    pltpu.sync_copy(w2_sh.at[pl.ds(sid * W2T, W2T)],
                    w2_out.at[cid, pl.ds(sid * W2T, W2T)])


_k3b = functools.partial(
    pl.kernel,
    out_type=jax.ShapeDtypeStruct((NCORES, W2SZ), jnp.float32),
    mesh=plsc.VectorSubcoreMesh(**_MESH),
    scratch_types=[
        pltpu.VMEM((C,), jnp.int32),
        pltpu.VMEM((C,), jnp.int32),
        pltpu.VMEM((C,), jnp.float32),
        pltpu.VMEM((C,), jnp.int32),
        pltpu.VMEM((C,), jnp.int32),
        pltpu.VMEM_SHARED((NPAD,), jnp.float32),
        pltpu.VMEM_SHARED((NPAD,), jnp.int32),
        pltpu.VMEM_SHARED((W2SZ,), jnp.float32),
    ],
    compiler_params=_SC_PARAMS,
)(_k3b_body)


BLK = 2048
NB = NPAD // BLK


def _k4_body(s2_ref, w2_ref, dinv_ref, w0_ref, b0_ref, w1_ref, b1_ref,
             cnt_ref, out_ref, acc_ref):
    i = pl.program_id(0)

    @pl.when(i == 0)
    def _():
        acc_ref[...] = jnp.zeros_like(acc_ref)

    s = s2_ref[0] + s2_ref[1]
    dv = dinv_ref[...]
    a = s * dv
    x1 = jnp.dot(a, w0_ref[...], preferred_element_type=jnp.float32)
    x1 = jnp.maximum(x1 + b0_ref[...], 0.0)
    nid = i * BLK + lax.broadcasted_iota(jnp.int32, (BLK, 1), 0)
    z = jnp.where(nid < N_NODES, x1 * dv, 0.0)
    t = w2_ref[0] + w2_ref[1]
    acc_ref[...] += jnp.dot(t, z, preferred_element_type=jnp.float32)

    @pl.when(i == NB - 1)
    def _():
        m = jnp.dot(acc_ref[...], w1_ref[...],
                    preferred_element_type=jnp.float32)
        rows = lax.broadcasted_iota(jnp.int32, (NUM_GRAPHS, OUT_CH), 0)
        den = jnp.ones((NUM_GRAPHS, OUT_CH), jnp.float32)
        for g in range(NUM_GRAPHS):
            den = jnp.where(rows == g, jnp.maximum(cnt_ref[0, g], 1.0), den)
        out_ref[...] = m / den + b1_ref[...]


def _k4(s2, w2, dinv, w0p, b0, w1, b1, cnt):
    return pl.pallas_call(
        _k4_body,
        grid=(NB,),
        in_specs=[
            pl.BlockSpec((NCORES, BLK, RW), lambda i: (0, i, 0)),
            pl.BlockSpec((NCORES, NUM_GRAPHS, BLK), lambda i: (0, 0, i)),
            pl.BlockSpec((BLK, 1), lambda i: (i, 0)),
            pl.BlockSpec((RW, HIDDEN), lambda i: (0, 0)),
            pl.BlockSpec((1, HIDDEN), lambda i: (0, 0)),
            pl.BlockSpec((HIDDEN, OUT_CH), lambda i: (0, 0)),
            pl.BlockSpec((1, OUT_CH), lambda i: (0, 0)),
            pl.BlockSpec(memory_space=pltpu.SMEM),
        ],
        out_specs=pl.BlockSpec((NUM_GRAPHS, OUT_CH), lambda i: (0, 0)),
        out_shape=jax.ShapeDtypeStruct((NUM_GRAPHS, OUT_CH), jnp.float32),
        scratch_shapes=[pltpu.VMEM((NUM_GRAPHS, HIDDEN), jnp.float32)],
        compiler_params=pltpu.CompilerParams(
            dimension_semantics=("arbitrary",)),
    )(s2, w2, dinv, w0p, b0, w1, b1, cnt)


def kernel(pos, edge_index, batch, W0, b0, W1, b1):
    src = edge_index[0].astype(jnp.int32)
    dst = edge_index[1].astype(jnp.int32)
    loop = jnp.arange(N_NODES, dtype=jnp.int32)
    npad_e = E_TOTAL - (N_EDGES + N_NODES)
    padidx = (N_NODES + (jnp.arange(npad_e, dtype=jnp.int32) % 64))
    srcp = jnp.concatenate([src, loop, padidx])
    dstp = jnp.concatenate([dst, loop, padidx])
    batchp = jnp.concatenate([
        batch.astype(jnp.int32),
        jnp.zeros((NPAD - N_NODES,), jnp.int32)])
    posi = (jnp.zeros((NPAD, RW), jnp.float32).at[:N_NODES, :IN_CH].set(pos)
            .reshape(NROW, 128 * RW))
    zs = jnp.zeros((NP16 // 4, RW), jnp.float32)
    zw = jnp.zeros((W2T,), jnp.float32)

    deg2 = _k1(dstp, zw)
    dinv2d, qi, cnt = _k2(deg2.reshape(NCORES, NROW, 128),
                          batchp.reshape(NROW, 128), posi)
    dinv = dinv2d.reshape(NPAD)
    s2 = _k3a(srcp, dstp, qi.reshape(NPAD, RW), zs)
    w2 = _k3b(srcp, dstp, dinv, batchp, zw)
    w2g = w2.reshape(NCORES, NUM_GRAPHS, NPAD)
    W0p = jnp.concatenate(
        [W0, jnp.zeros((RW - IN_CH, HIDDEN), jnp.float32)], axis=0)
    return _k4(s2, w2g, dinv.reshape(NPAD, 1), W0p, b0.reshape(1, HIDDEN),
               W1, b1.reshape(1, OUT_CH), cnt)


# speedup vs baseline: 92.2769x; 1.1105x over previous
import functools

import jax
import jax.numpy as jnp
from jax import lax
from jax.experimental import pallas as pl
from jax.experimental.pallas import tpu as pltpu
from jax.experimental.pallas import tpu_sc as plsc

N_NODES = 50000
N_EDGES = 1600000
NUM_GRAPHS = 16
IN_CH = 3
HIDDEN = 64
OUT_CH = 24

NPAD = 51200
NROW = NPAD // 128
NCORES = 2
NSUB = 16
NTILES = NCORES * NSUB
C = 1024
CHUNKS = 51
E_TILE = C * CHUNKS
E_TOTAL = E_TILE * NTILES
NP16 = NPAD // NSUB
W2SZ = NUM_GRAPHS * NPAD
W2T = W2SZ // NSUB
RW = 8

_MESH = dict(core_axis_name="c", subcore_axis_name="s",
             num_cores=NCORES, num_subcores=NSUB)
_SC_PARAMS = pltpu.CompilerParams(needs_layout_passes=False,
                                  use_tc_tiling_on_sc=False)


def _wid():
    return lax.axis_index("c") * NSUB + lax.axis_index("s")


def _k1_body(dst_hbm, zw_hbm, deg_out, dst_v, ones_v, deg_sh):
    cid = lax.axis_index("c")
    sid = lax.axis_index("s")
    w = _wid()
    for g in range(C // 16):
        ones_v[pl.ds(g * 16, 16)] = jnp.ones((16,), jnp.float32)
    pltpu.sync_copy(zw_hbm.at[pl.ds(0, NP16)],
                    deg_sh.at[pl.ds(sid * NP16, NP16)])
    plsc.subcore_barrier()

    def chunk(i, _):
        eb = pl.multiple_of(w * E_TILE + i * C, C)
        pltpu.sync_copy(dst_hbm.at[pl.ds(eb, C)], dst_v)
        pltpu.sync_copy(ones_v, deg_sh.at[dst_v], add=True)
        return ()

    lax.fori_loop(0, CHUNKS, chunk, ())
    plsc.subcore_barrier()
    pltpu.sync_copy(deg_sh.at[pl.ds(sid * NP16, NP16)],
                    deg_out.at[cid, pl.ds(sid * NP16, NP16)])


_k1 = functools.partial(
    pl.kernel,
    out_type=jax.ShapeDtypeStruct((NCORES, NPAD), jnp.float32),
    mesh=plsc.VectorSubcoreMesh(**_MESH),
    scratch_types=[
        pltpu.VMEM((C,), jnp.int32),
        pltpu.VMEM((C,), jnp.float32),
        pltpu.VMEM_SHARED((NPAD,), jnp.float32),
    ],
    compiler_params=_SC_PARAMS,
)(_k1_body)


def _k2_body(deg2_ref, batch_ref, posi_ref, dinv_ref, qi_ref, combo_ref,
             cnt_ref):
    deg = deg2_ref[0] + deg2_ref[1]
    dinv = jnp.where(deg > 0, lax.rsqrt(deg), 0.0)
    dinv_ref[...] = dinv
    qi_ref[...] = jnp.repeat(dinv, RW, axis=1) * posi_ref[...]
    b = batch_ref[...]
    bits = lax.bitcast_convert_type(dinv, jnp.int32)
    combo_ref[...] = (bits & ~jnp.int32(15)) | b
    nid = (lax.broadcasted_iota(jnp.int32, (NROW, 128), 0) * 128
           + lax.broadcasted_iota(jnp.int32, (NROW, 128), 1))
    real = nid < N_NODES
    for g in range(NUM_GRAPHS):
        cnt_ref[0, g] = jnp.sum(jnp.where((b == g) & real, 1.0, 0.0))


def _k2(deg2, batch2d, posi):
    return pl.pallas_call(
        _k2_body,
        out_shape=[
            jax.ShapeDtypeStruct((NROW, 128), jnp.float32),
            jax.ShapeDtypeStruct((NROW, 128 * RW), jnp.float32),
            jax.ShapeDtypeStruct((NROW, 128), jnp.int32),
            jax.ShapeDtypeStruct((1, NUM_GRAPHS), jnp.float32),
        ],
        out_specs=[
            pl.BlockSpec(memory_space=pltpu.VMEM),
            pl.BlockSpec(memory_space=pltpu.VMEM),
            pl.BlockSpec(memory_space=pltpu.VMEM),
            pl.BlockSpec(memory_space=pltpu.SMEM),
        ],
    )(deg2, batch2d, posi)


def _k3_body(src_hbm, dst_hbm, q_hbm, combo_hbm, zs_hbm, zw_hbm,
             s_out, w2_out,
             src_v, dst_v, cb_v, val_v, idx_v, qrow_v,
             q_sh, s_sh, combo_sh, w2_sh):
    cid = lax.axis_index("c")
    sid = lax.axis_index("s")
    w = _wid()

    pltpu.sync_copy(q_hbm.at[pl.ds(sid * NP16, NP16)],
                    q_sh.at[pl.ds(sid * NP16, NP16)])
    pltpu.sync_copy(combo_hbm.at[pl.ds(sid * NP16, NP16)],
                    combo_sh.at[pl.ds(sid * NP16, NP16)])
    for r in range(4):
        pltpu.sync_copy(zs_hbm,
                        s_sh.at[pl.ds(sid * NP16 + r * (NP16 // 4), NP16 // 4)])
    pltpu.sync_copy(zw_hbm, w2_sh.at[pl.ds(sid * W2T, W2T)])
    plsc.subcore_barrier()

    def chunk(i, _):
        eb = pl.multiple_of(w * E_TILE + i * C, C)
        pltpu.sync_copy(src_hbm.at[pl.ds(eb, C)], src_v)
        pltpu.sync_copy(dst_hbm.at[pl.ds(eb, C)], dst_v)
        pltpu.sync_copy(q_sh.at[src_v], qrow_v)
        pltpu.sync_copy(qrow_v, s_sh.at[dst_v], add=True)
        pltpu.sync_copy(combo_sh.at[dst_v], cb_v)
        for g in range(C // 16):
            cb = cb_v[pl.ds(g * 16, 16)]
            gid = cb & 15
            val_v[pl.ds(g * 16, 16)] = plsc.bitcast(cb & ~jnp.int32(15),
                                                    jnp.float32)
            idx_v[pl.ds(g * 16, 16)] = gid * NPAD + src_v[pl.ds(g * 16, 16)]
        pltpu.sync_copy(val_v, w2_sh.at[idx_v], add=True)
        return ()

    lax.fori_loop(0, CHUNKS, chunk, ())
    plsc.subcore_barrier()
    pltpu.sync_copy(s_sh.at[pl.ds(sid * NP16, NP16)],
                    s_out.at[cid, pl.ds(sid * NP16, NP16)])
    pltpu.sync_copy(w2_sh.at[pl.ds(sid * W2T, W2T)],
                    w2_out.at[cid, pl.ds(sid * W2T, W2T)])


_k3 = functools.partial(
    pl.kernel,
    out_type=[
        jax.ShapeDtypeStruct((NCORES, NPAD, RW), jnp.float32),
        jax.ShapeDtypeStruct((NCORES, W2SZ), jnp.float32),
    ],
    mesh=plsc.VectorSubcoreMesh(**_MESH),
    scratch_types=[
        pltpu.VMEM((C,), jnp.int32),
        pltpu.VMEM((C,), jnp.int32),
        pltpu.VMEM((C,), jnp.int32),
        pltpu.VMEM((C,), jnp.float32),
        pltpu.VMEM((C,), jnp.int32),
        pltpu.VMEM((C, RW), jnp.float32),
        pltpu.VMEM_SHARED((NPAD, RW), jnp.float32),
        pltpu.VMEM_SHARED((NPAD, RW), jnp.float32),
        pltpu.VMEM_SHARED((NPAD,), jnp.int32),
        pltpu.VMEM_SHARED((W2SZ,), jnp.float32),
    ],
    compiler_params=_SC_PARAMS,
)(_k3_body)


BLK = 2048
NB = NPAD // BLK


def _k4_body(s2_ref, w2_ref, dinv_ref, w0_ref, b0_ref, w1_ref, b1_ref,
             cnt_ref, out_ref, acc_ref):
    i = pl.program_id(0)

    @pl.when(i == 0)
    def _():
        acc_ref[...] = jnp.zeros_like(acc_ref)

    s = s2_ref[0] + s2_ref[1]
    dv = dinv_ref[...]
    a = s * dv
    x1 = jnp.dot(a, w0_ref[...], preferred_element_type=jnp.float32)
    x1 = jnp.maximum(x1 + b0_ref[...], 0.0)
    nid = i * BLK + lax.broadcasted_iota(jnp.int32, (BLK, 1), 0)
    z = jnp.where(nid < N_NODES, x1 * dv, 0.0)
    t = w2_ref[0] + w2_ref[1]
    acc_ref[...] += jnp.dot(t, z, preferred_element_type=jnp.float32)

    @pl.when(i == NB - 1)
    def _():
        m = jnp.dot(acc_ref[...], w1_ref[...],
                    preferred_element_type=jnp.float32)
        rows = lax.broadcasted_iota(jnp.int32, (NUM_GRAPHS, OUT_CH), 0)
        den = jnp.ones((NUM_GRAPHS, OUT_CH), jnp.float32)
        for g in range(NUM_GRAPHS):
            den = jnp.where(rows == g, jnp.maximum(cnt_ref[0, g], 1.0), den)
        out_ref[...] = m / den + b1_ref[...]


def _k4(s2, w2, dinv, w0p, b0, w1, b1, cnt):
    return pl.pallas_call(
        _k4_body,
        grid=(NB,),
        in_specs=[
            pl.BlockSpec((NCORES, BLK, RW), lambda i: (0, i, 0)),
            pl.BlockSpec((NCORES, NUM_GRAPHS, BLK), lambda i: (0, 0, i)),
            pl.BlockSpec((BLK, 1), lambda i: (i, 0)),
            pl.BlockSpec((RW, HIDDEN), lambda i: (0, 0)),
            pl.BlockSpec((1, HIDDEN), lambda i: (0, 0)),
            pl.BlockSpec((HIDDEN, OUT_CH), lambda i: (0, 0)),
            pl.BlockSpec((1, OUT_CH), lambda i: (0, 0)),
            pl.BlockSpec(memory_space=pltpu.SMEM),
        ],
        out_specs=pl.BlockSpec((NUM_GRAPHS, OUT_CH), lambda i: (0, 0)),
        out_shape=jax.ShapeDtypeStruct((NUM_GRAPHS, OUT_CH), jnp.float32),
        scratch_shapes=[pltpu.VMEM((NUM_GRAPHS, HIDDEN), jnp.float32)],
        compiler_params=pltpu.CompilerParams(
            dimension_semantics=("arbitrary",)),
    )(s2, w2, dinv, w0p, b0, w1, b1, cnt)


def kernel(pos, edge_index, batch, W0, b0, W1, b1):
    src = edge_index[0].astype(jnp.int32)
    dst = edge_index[1].astype(jnp.int32)
    loop = jnp.arange(N_NODES, dtype=jnp.int32)
    npad_e = E_TOTAL - (N_EDGES + N_NODES)
    padidx = (N_NODES + (jnp.arange(npad_e, dtype=jnp.int32) % 64))
    srcp = jnp.concatenate([src, loop, padidx])
    dstp = jnp.concatenate([dst, loop, padidx])
    batchp = jnp.concatenate([
        batch.astype(jnp.int32),
        jnp.zeros((NPAD - N_NODES,), jnp.int32)])
    posi = (jnp.zeros((NPAD, RW), jnp.float32).at[:N_NODES, :IN_CH].set(pos)
            .reshape(NROW, 128 * RW))
    zs = jnp.zeros((NP16 // 4, RW), jnp.float32)
    zw = jnp.zeros((W2T,), jnp.float32)

    deg2 = _k1(dstp, zw)
    dinv2d, qi, combo, cnt = _k2(deg2.reshape(NCORES, NROW, 128),
                                 batchp.reshape(NROW, 128), posi)
    dinv = dinv2d.reshape(NPAD)
    s2, w2 = _k3(srcp, dstp, qi.reshape(NPAD, RW), combo.reshape(NPAD),
                 zs, zw)
    w2g = w2.reshape(NCORES, NUM_GRAPHS, NPAD)
    W0p = jnp.concatenate(
        [W0, jnp.zeros((RW - IN_CH, HIDDEN), jnp.float32)], axis=0)
    return _k4(s2, w2g, dinv.reshape(NPAD, 1), W0p, b0.reshape(1, HIDDEN),
               W1, b1.reshape(1, OUT_CH), cnt)


# speedup vs baseline: 101.5719x; 1.1007x over previous
import functools

import jax
import jax.numpy as jnp
from jax import lax
from jax.experimental import pallas as pl
from jax.experimental.pallas import tpu as pltpu
from jax.experimental.pallas import tpu_sc as plsc

N_NODES = 50000
N_EDGES = 1600000
NUM_GRAPHS = 16
IN_CH = 3
HIDDEN = 64
OUT_CH = 24

NPAD = 51200
NROW = NPAD // 128
NCORES = 2
NSUB = 16
NTILES = NCORES * NSUB
C = 1024
CHUNKS = 51
E_TILE = C * CHUNKS
E_TOTAL = E_TILE * NTILES
NP16 = NPAD // NSUB
W2SZ = NUM_GRAPHS * NPAD
W2T = W2SZ // NSUB
RW = 8

_MESH = dict(core_axis_name="c", subcore_axis_name="s",
             num_cores=NCORES, num_subcores=NSUB)
_SC_PARAMS = pltpu.CompilerParams(needs_layout_passes=False,
                                  use_tc_tiling_on_sc=False)


def _wid():
    return lax.axis_index("c") * NSUB + lax.axis_index("s")


def _k1_body(dst_hbm, zw_hbm, deg_out, dst_v, ones_v, deg_sh):
    cid = lax.axis_index("c")
    sid = lax.axis_index("s")
    w = _wid()
    for g in range(C // 16):
        ones_v[pl.ds(g * 16, 16)] = jnp.ones((16,), jnp.float32)
    pltpu.sync_copy(zw_hbm.at[pl.ds(0, NP16)],
                    deg_sh.at[pl.ds(sid * NP16, NP16)])
    plsc.subcore_barrier()

    def chunk(i, _):
        eb = pl.multiple_of(w * E_TILE + i * C, C)
        pltpu.sync_copy(dst_hbm.at[pl.ds(eb, C)], dst_v)
        pltpu.sync_copy(ones_v, deg_sh.at[dst_v], add=True)
        return ()

    lax.fori_loop(0, CHUNKS, chunk, ())
    plsc.subcore_barrier()
    pltpu.sync_copy(deg_sh.at[pl.ds(sid * NP16, NP16)],
                    deg_out.at[cid, pl.ds(sid * NP16, NP16)])


_k1 = functools.partial(
    pl.kernel,
    out_type=jax.ShapeDtypeStruct((NCORES, NPAD), jnp.float32),
    mesh=plsc.VectorSubcoreMesh(**_MESH),
    scratch_types=[
        pltpu.VMEM((C,), jnp.int32),
        pltpu.VMEM((C,), jnp.float32),
        pltpu.VMEM_SHARED((NPAD,), jnp.float32),
    ],
    compiler_params=_SC_PARAMS,
)(_k1_body)


def _k2_body(deg2_ref, batch_ref, posi_ref, dinv_ref, qi_ref, combo_ref,
             cnt_ref):
    deg = deg2_ref[0] + deg2_ref[1]
    dinv = jnp.where(deg > 0, lax.rsqrt(deg), 0.0)
    dinv_ref[...] = dinv
    qi_ref[...] = jnp.repeat(dinv, RW, axis=1) * posi_ref[...]
    b = batch_ref[...]
    bits = lax.bitcast_convert_type(dinv, jnp.int32)
    combo_ref[...] = (bits & ~jnp.int32(15)) | b
    nid = (lax.broadcasted_iota(jnp.int32, (NROW, 128), 0) * 128
           + lax.broadcasted_iota(jnp.int32, (NROW, 128), 1))
    real = nid < N_NODES
    for g in range(NUM_GRAPHS):
        cnt_ref[0, g] = jnp.sum(jnp.where((b == g) & real, 1.0, 0.0))


def _k2(deg2, batch2d, posi):
    return pl.pallas_call(
        _k2_body,
        out_shape=[
            jax.ShapeDtypeStruct((NROW, 128), jnp.float32),
            jax.ShapeDtypeStruct((NROW, 128 * RW), jnp.float32),
            jax.ShapeDtypeStruct((NROW, 128), jnp.int32),
            jax.ShapeDtypeStruct((1, NUM_GRAPHS), jnp.float32),
        ],
        out_specs=[
            pl.BlockSpec(memory_space=pltpu.VMEM),
            pl.BlockSpec(memory_space=pltpu.VMEM),
            pl.BlockSpec(memory_space=pltpu.VMEM),
            pl.BlockSpec(memory_space=pltpu.SMEM),
        ],
    )(deg2, batch2d, posi)


def _k3_body(src_hbm, dst_hbm, q_hbm, combo_hbm, zs_hbm, zw_hbm,
             s_out, w2_out,
             src_v, dst_v, cb_v, val_v, idx_v, qrow_v,
             q_sh, s_sh, combo_sh, w2_sh, sem_a, sem_b):
    cid = lax.axis_index("c")
    sid = lax.axis_index("s")
    w = _wid()

    pltpu.sync_copy(q_hbm.at[pl.ds(sid * NP16, NP16)],
                    q_sh.at[pl.ds(sid * NP16, NP16)])
    pltpu.sync_copy(combo_hbm.at[pl.ds(sid * NP16, NP16)],
                    combo_sh.at[pl.ds(sid * NP16, NP16)])
    for r in range(4):
        pltpu.sync_copy(zs_hbm,
                        s_sh.at[pl.ds(sid * NP16 + r * (NP16 // 4), NP16 // 4)])
    pltpu.sync_copy(zw_hbm, w2_sh.at[pl.ds(sid * W2T, W2T)])
    plsc.subcore_barrier()

    def chunk(i, _):
        eb = pl.multiple_of(w * E_TILE + i * C, C)
        c1 = pltpu.async_copy(src_hbm.at[pl.ds(eb, C)], src_v, sem_a)
        c2 = pltpu.async_copy(dst_hbm.at[pl.ds(eb, C)], dst_v, sem_b)
        c1.wait()
        c2.wait()
        g1 = pltpu.async_copy(q_sh.at[src_v], qrow_v, sem_a)
        g2 = pltpu.async_copy(combo_sh.at[dst_v], cb_v, sem_b)
        g2.wait()
        for g in range(C // 16):
            cb = cb_v[pl.ds(g * 16, 16)]
            gid = cb & 15
            val_v[pl.ds(g * 16, 16)] = plsc.bitcast(cb & ~jnp.int32(15),
                                                    jnp.float32)
            idx_v[pl.ds(g * 16, 16)] = gid * NPAD + src_v[pl.ds(g * 16, 16)]
        g1.wait()
        s1 = pltpu.async_copy(qrow_v, s_sh.at[dst_v], sem_a, add=True)
        s2 = pltpu.async_copy(val_v, w2_sh.at[idx_v], sem_b, add=True)
        s1.wait()
        s2.wait()
        return ()

    lax.fori_loop(0, CHUNKS, chunk, ())
    plsc.subcore_barrier()
    pltpu.sync_copy(s_sh.at[pl.ds(sid * NP16, NP16)],
                    s_out.at[cid, pl.ds(sid * NP16, NP16)])
    pltpu.sync_copy(w2_sh.at[pl.ds(sid * W2T, W2T)],
                    w2_out.at[cid, pl.ds(sid * W2T, W2T)])


_k3 = functools.partial(
    pl.kernel,
    out_type=[
        jax.ShapeDtypeStruct((NCORES, NPAD, RW), jnp.float32),
        jax.ShapeDtypeStruct((NCORES, W2SZ), jnp.float32),
    ],
    mesh=plsc.VectorSubcoreMesh(**_MESH),
    scratch_types=[
        pltpu.VMEM((C,), jnp.int32),
        pltpu.VMEM((C,), jnp.int32),
        pltpu.VMEM((C,), jnp.int32),
        pltpu.VMEM((C,), jnp.float32),
        pltpu.VMEM((C,), jnp.int32),
        pltpu.VMEM((C, RW), jnp.float32),
        pltpu.VMEM_SHARED((NPAD, RW), jnp.float32),
        pltpu.VMEM_SHARED((NPAD, RW), jnp.float32),
        pltpu.VMEM_SHARED((NPAD,), jnp.int32),
        pltpu.VMEM_SHARED((W2SZ,), jnp.float32),
        pltpu.SemaphoreType.DMA,
        pltpu.SemaphoreType.DMA,
    ],
    compiler_params=_SC_PARAMS,
)(_k3_body)


BLK = 2048
NB = NPAD // BLK


def _k4_body(s2_ref, w2_ref, dinv_ref, w0_ref, b0_ref, w1_ref, b1_ref,
             cnt_ref, out_ref, acc_ref):
    i = pl.program_id(0)

    @pl.when(i == 0)
    def _():
        acc_ref[...] = jnp.zeros_like(acc_ref)

    s = s2_ref[0] + s2_ref[1]
    dv = dinv_ref[...]
    a = s * dv
    x1 = jnp.dot(a, w0_ref[...], preferred_element_type=jnp.float32)
    x1 = jnp.maximum(x1 + b0_ref[...], 0.0)
    nid = i * BLK + lax.broadcasted_iota(jnp.int32, (BLK, 1), 0)
    z = jnp.where(nid < N_NODES, x1 * dv, 0.0)
    t = w2_ref[0] + w2_ref[1]
    acc_ref[...] += jnp.dot(t, z, preferred_element_type=jnp.float32)

    @pl.when(i == NB - 1)
    def _():
        m = jnp.dot(acc_ref[...], w1_ref[...],
                    preferred_element_type=jnp.float32)
        rows = lax.broadcasted_iota(jnp.int32, (NUM_GRAPHS, OUT_CH), 0)
        den = jnp.ones((NUM_GRAPHS, OUT_CH), jnp.float32)
        for g in range(NUM_GRAPHS):
            den = jnp.where(rows == g, jnp.maximum(cnt_ref[0, g], 1.0), den)
        out_ref[...] = m / den + b1_ref[...]


def _k4(s2, w2, dinv, w0p, b0, w1, b1, cnt):
    return pl.pallas_call(
        _k4_body,
        grid=(NB,),
        in_specs=[
            pl.BlockSpec((NCORES, BLK, RW), lambda i: (0, i, 0)),
            pl.BlockSpec((NCORES, NUM_GRAPHS, BLK), lambda i: (0, 0, i)),
            pl.BlockSpec((BLK, 1), lambda i: (i, 0)),
            pl.BlockSpec((RW, HIDDEN), lambda i: (0, 0)),
            pl.BlockSpec((1, HIDDEN), lambda i: (0, 0)),
            pl.BlockSpec((HIDDEN, OUT_CH), lambda i: (0, 0)),
            pl.BlockSpec((1, OUT_CH), lambda i: (0, 0)),
            pl.BlockSpec(memory_space=pltpu.SMEM),
        ],
        out_specs=pl.BlockSpec((NUM_GRAPHS, OUT_CH), lambda i: (0, 0)),
        out_shape=jax.ShapeDtypeStruct((NUM_GRAPHS, OUT_CH), jnp.float32),
        scratch_shapes=[pltpu.VMEM((NUM_GRAPHS, HIDDEN), jnp.float32)],
        compiler_params=pltpu.CompilerParams(
            dimension_semantics=("arbitrary",)),
    )(s2, w2, dinv, w0p, b0, w1, b1, cnt)


def kernel(pos, edge_index, batch, W0, b0, W1, b1):
    src = edge_index[0].astype(jnp.int32)
    dst = edge_index[1].astype(jnp.int32)
    loop = jnp.arange(N_NODES, dtype=jnp.int32)
    npad_e = E_TOTAL - (N_EDGES + N_NODES)
    padidx = (N_NODES + (jnp.arange(npad_e, dtype=jnp.int32) % 64))
    srcp = jnp.concatenate([src, loop, padidx])
    dstp = jnp.concatenate([dst, loop, padidx])
    batchp = jnp.concatenate([
        batch.astype(jnp.int32),
        jnp.zeros((NPAD - N_NODES,), jnp.int32)])
    posi = (jnp.zeros((NPAD, RW), jnp.float32).at[:N_NODES, :IN_CH].set(pos)
            .reshape(NROW, 128 * RW))
    zs = jnp.zeros((NP16 // 4, RW), jnp.float32)
    zw = jnp.zeros((W2T,), jnp.float32)

    deg2 = _k1(dstp, zw)
    dinv2d, qi, combo, cnt = _k2(deg2.reshape(NCORES, NROW, 128),
                                 batchp.reshape(NROW, 128), posi)
    dinv = dinv2d.reshape(NPAD)
    s2, w2 = _k3(srcp, dstp, qi.reshape(NPAD, RW), combo.reshape(NPAD),
                 zs, zw)
    w2g = w2.reshape(NCORES, NUM_GRAPHS, NPAD)
    W0p = jnp.concatenate(
        [W0, jnp.zeros((RW - IN_CH, HIDDEN), jnp.float32)], axis=0)
    return _k4(s2, w2g, dinv.reshape(NPAD, 1), W0p, b0.reshape(1, HIDDEN),
               W1, b1.reshape(1, OUT_CH), cnt)
